# Initial kernel scaffold; baseline (speedup 1.0000x reference)
import jax
import jax.numpy as jnp
from jax.experimental import pallas as pl


def kernel(heads, relations, tails, entity_table, relation_table):
    raise NotImplementedError("write your pallas kernel here")


# speedup vs baseline: 2.1250x; 2.1250x over previous
import functools

import jax
import jax.numpy as jnp
from jax import lax
from jax.experimental import pallas as pl
from jax.experimental.pallas import tpu as pltpu
from jax.experimental.pallas import tpu_sc as plsc

B = 16384
D = 128
NC = 2
NS = 16
NW = NC * NS
BPW = B // NW
C = 128
NCH = BPW // C
LANES = 16


def _body(heads_hbm, rels_hbm, tails_hbm, ent_hbm, rel_hbm, out_hbm,
          hidx, ridx, tidx, hrows, rrows, trows, orows,
          sem_h, sem_r, sem_t):
    wid = lax.axis_index("s") * NC + lax.axis_index("c")
    base = wid * BPW

    pltpu.sync_copy(heads_hbm.at[pl.ds(base, BPW)], hidx)
    pltpu.sync_copy(rels_hbm.at[pl.ds(base, BPW)], ridx)
    pltpu.sync_copy(tails_hbm.at[pl.ds(base, BPW)], tidx)

    def chunk(g, carry):
        off = g * C
        ch = pltpu.async_copy(ent_hbm.at[hidx.at[pl.ds(off, C)]], hrows, sem_h)
        cr = pltpu.async_copy(rel_hbm.at[ridx.at[pl.ds(off, C)]], rrows, sem_r)
        ct = pltpu.async_copy(ent_hbm.at[tidx.at[pl.ds(off, C)]], trows, sem_t)
        ch.wait()
        cr.wait()
        ct.wait()

        def row(r, rc):
            for d in range(D // LANES):
                sl = pl.ds(d * LANES, LANES)
                orows[r, sl] = hrows[r, sl] + rrows[r, sl] - trows[r, sl]
            return rc

        lax.fori_loop(0, C, row, 0)
        pltpu.sync_copy(orows, out_hbm.at[pl.ds(base + off, C)])
        return carry

    lax.fori_loop(0, NCH, chunk, 0)


def kernel(heads, relations, tails, entity_table, relation_table):
    mesh = plsc.VectorSubcoreMesh(core_axis_name="c", subcore_axis_name="s")
    k = functools.partial(
        pl.kernel,
        mesh=mesh,
        out_type=jax.ShapeDtypeStruct((B, D), jnp.float32),
        scratch_types=[
            pltpu.VMEM((BPW,), jnp.int32),
            pltpu.VMEM((BPW,), jnp.int32),
            pltpu.VMEM((BPW,), jnp.int32),
            pltpu.VMEM((C, D), jnp.float32),
            pltpu.VMEM((C, D), jnp.float32),
            pltpu.VMEM((C, D), jnp.float32),
            pltpu.VMEM((C, D), jnp.float32),
            pltpu.SemaphoreType.DMA,
            pltpu.SemaphoreType.DMA,
            pltpu.SemaphoreType.DMA,
        ],
    )(_body)
    return k(heads.astype(jnp.int32), relations.astype(jnp.int32),
             tails.astype(jnp.int32), entity_table, relation_table)


# speedup vs baseline: 2.3895x; 1.1245x over previous
import functools

import jax
import jax.numpy as jnp
from jax import lax
from jax.experimental import pallas as pl
from jax.experimental.pallas import tpu as pltpu
from jax.experimental.pallas import tpu_sc as plsc

B = 16384
D = 128
NC = 2
NS = 16
NW = NC * NS
BPW = B // NW
C = 64
NCH = BPW // C
LANES = 16


def _body(heads_hbm, rels_hbm, tails_hbm, ent_hbm, rel_hbm, out_hbm,
          hidx, ridx, tidx, hb, rb, tb, ob,
          sem_g0, sem_g1, sem_o0, sem_o1):
    wid = lax.axis_index("s") * NC + lax.axis_index("c")
    base = wid * BPW

    pltpu.sync_copy(heads_hbm.at[pl.ds(base, BPW)], hidx)
    pltpu.sync_copy(rels_hbm.at[pl.ds(base, BPW)], ridx)
    pltpu.sync_copy(tails_hbm.at[pl.ds(base, BPW)], tidx)

    sg = (sem_g0, sem_g1)
    so = (sem_o0, sem_o1)

    def start_gather(g, p):
        off = g * C
        return (
            pltpu.async_copy(ent_hbm.at[hidx.at[pl.ds(off, C)]], hb.at[p], sg[p]),
            pltpu.async_copy(rel_hbm.at[ridx.at[pl.ds(off, C)]], rb.at[p], sg[p]),
            pltpu.async_copy(ent_hbm.at[tidx.at[pl.ds(off, C)]], tb.at[p], sg[p]),
        )

    inflight = [None, None]
    outflight = [None, None]
    inflight[0] = start_gather(0, 0)

    for g in range(NCH):
        p = g & 1
        if g + 1 < NCH:
            inflight[1 - p] = start_gather(g + 1, 1 - p)
        for cp in inflight[p]:
            cp.wait()
        if outflight[p] is not None:
            outflight[p].wait()

        hp, rp, tp, op = hb.at[p], rb.at[p], tb.at[p], ob.at[p]

        def row(r, rc):
            for d in range(D // LANES):
                sl = pl.ds(d * LANES, LANES)
                op[r, sl] = hp[r, sl] + rp[r, sl] - tp[r, sl]
            return rc

        lax.fori_loop(0, C, row, 0)
        outflight[p] = pltpu.async_copy(
            op, out_hbm.at[pl.ds(base + g * C, C)], so[p])

    for p in range(2):
        if outflight[p] is not None:
            outflight[p].wait()


def kernel(heads, relations, tails, entity_table, relation_table):
    mesh = plsc.VectorSubcoreMesh(core_axis_name="c", subcore_axis_name="s")
    k = functools.partial(
        pl.kernel,
        mesh=mesh,
        out_type=jax.ShapeDtypeStruct((B, D), jnp.float32),
        scratch_types=[
            pltpu.VMEM((BPW,), jnp.int32),
            pltpu.VMEM((BPW,), jnp.int32),
            pltpu.VMEM((BPW,), jnp.int32),
            pltpu.VMEM((2, C, D), jnp.float32),
            pltpu.VMEM((2, C, D), jnp.float32),
            pltpu.VMEM((2, C, D), jnp.float32),
            pltpu.VMEM((2, C, D), jnp.float32),
            pltpu.SemaphoreType.DMA,
            pltpu.SemaphoreType.DMA,
            pltpu.SemaphoreType.DMA,
            pltpu.SemaphoreType.DMA,
        ],
    )(_body)
    return k(heads.astype(jnp.int32), relations.astype(jnp.int32),
             tails.astype(jnp.int32), entity_table, relation_table)


# speedup vs baseline: 2.4861x; 1.0405x over previous
import functools

import jax
import jax.numpy as jnp
from jax import lax
from jax.experimental import pallas as pl
from jax.experimental.pallas import tpu as pltpu
from jax.experimental.pallas import tpu_sc as plsc

B = 16384
D = 128
NC = 2
NS = 16
NW = NC * NS
BPW = B // NW
C = 64
NCH = BPW // C
LANES = 16


def _body(heads_hbm, rels_hbm, tails_hbm, ent_hbm, rel_hbm, out_hbm,
          hidx, ridx, tidx, hb, rb, tb, ob, rel_sh,
          sem_g0, sem_g1, sem_o0, sem_o1, sem_r0, sem_r1):
    sid = lax.axis_index("s")
    wid = sid * NC + lax.axis_index("c")
    base = wid * BPW

    @pl.when(sid == 0)
    def _():
        pltpu.sync_copy(rel_hbm, rel_sh)

    pltpu.sync_copy(heads_hbm.at[pl.ds(base, BPW)], hidx)
    pltpu.sync_copy(rels_hbm.at[pl.ds(base, BPW)], ridx)
    pltpu.sync_copy(tails_hbm.at[pl.ds(base, BPW)], tidx)
    plsc.subcore_barrier()

    sg = (sem_g0, sem_g1)
    so = (sem_o0, sem_o1)
    sr = (sem_r0, sem_r1)

    def start_gather(g, p):
        off = g * C
        return (
            pltpu.async_copy(ent_hbm.at[hidx.at[pl.ds(off, C)]], hb.at[p], sg[p]),
            pltpu.async_copy(rel_sh.at[ridx.at[pl.ds(off, C)]], rb.at[p], sr[p]),
            pltpu.async_copy(ent_hbm.at[tidx.at[pl.ds(off, C)]], tb.at[p], sg[p]),
        )

    inflight = [None, None]
    outflight = [None, None]
    inflight[0] = start_gather(0, 0)

    for g in range(NCH):
        p = g & 1
        if g + 1 < NCH:
            inflight[1 - p] = start_gather(g + 1, 1 - p)
        for cp in inflight[p]:
            cp.wait()
        if outflight[p] is not None:
            outflight[p].wait()

        hp, rp, tp, op = hb.at[p], rb.at[p], tb.at[p], ob.at[p]

        def row(r, rc):
            for d in range(D // LANES):
                sl = pl.ds(d * LANES, LANES)
                op[r, sl] = hp[r, sl] + rp[r, sl] - tp[r, sl]
            return rc

        lax.fori_loop(0, C, row, 0)
        outflight[p] = pltpu.async_copy(
            op, out_hbm.at[pl.ds(base + g * C, C)], so[p])

    for p in range(2):
        if outflight[p] is not None:
            outflight[p].wait()


def kernel(heads, relations, tails, entity_table, relation_table):
    mesh = plsc.VectorSubcoreMesh(core_axis_name="c", subcore_axis_name="s")
    k = functools.partial(
        pl.kernel,
        mesh=mesh,
        out_type=jax.ShapeDtypeStruct((B, D), jnp.float32),
        scratch_types=[
            pltpu.VMEM((BPW,), jnp.int32),
            pltpu.VMEM((BPW,), jnp.int32),
            pltpu.VMEM((BPW,), jnp.int32),
            pltpu.VMEM((2, C, D), jnp.float32),
            pltpu.VMEM((2, C, D), jnp.float32),
            pltpu.VMEM((2, C, D), jnp.float32),
            pltpu.VMEM((2, C, D), jnp.float32),
            pltpu.VMEM_SHARED((1000, D), jnp.float32),
            pltpu.SemaphoreType.DMA,
            pltpu.SemaphoreType.DMA,
            pltpu.SemaphoreType.DMA,
            pltpu.SemaphoreType.DMA,
            pltpu.SemaphoreType.DMA,
            pltpu.SemaphoreType.DMA,
        ],
    )(_body)
    return k(heads.astype(jnp.int32), relations.astype(jnp.int32),
             tails.astype(jnp.int32), entity_table, relation_table)
